# baseline (speedup 1.0000x reference)
import jax
import jax.numpy as jnp
from jax.experimental import pallas as pl
from jax.experimental.pallas import tpu as pltpu

_F8 = jnp.float8_e4m3fn


def _layer1_kernel(x_ref, w1_ref, b1_ref, w2_ref, adj_ref,
                   hw2_ref, qadj_ref, xw1_scr):
    @pl.when(pl.program_id(0) == 0)
    def _():
        xw1 = jnp.dot(
            x_ref[...].astype(jnp.bfloat16),
            w1_ref[...].astype(jnp.bfloat16),
            preferred_element_type=jnp.float32,
        )
        xw1_scr[...] = xw1.astype(jnp.bfloat16)

    a = adj_ref[...]
    qadj_ref[...] = a.astype(_F8)

    h = (
        jnp.dot(
            a.astype(jnp.bfloat16),
            xw1_scr[...],
            preferred_element_type=jnp.float32,
        )
        + b1_ref[...]
    )
    h = jnp.maximum(h, 0.0)
    hw2s = jnp.dot(
        h.astype(jnp.bfloat16),
        w2_ref[...].astype(jnp.bfloat16),
        preferred_element_type=jnp.float32,
    )
    hw2_ref[...] = hw2s.astype(_F8)


def _layer2_kernel(hw2_ref, b2_ref, qadj_ref, out_ref):
    out_ref[...] = (
        jnp.dot(
            qadj_ref[...],
            hw2_ref[...],
            preferred_element_type=jnp.float32,
        )
        + b2_ref[...]
    )


def kernel(x, adj, W1, b1, W2, b2):
    n, c = x.shape
    hid = W1.shape[1]
    out_dim = W2.shape[1]
    bm1 = 400
    bm2 = 2000

    hw2s, qadj = pl.pallas_call(
        _layer1_kernel,
        grid=(n // bm1,),
        in_specs=[
            pl.BlockSpec((n, c), lambda i: (0, 0)),
            pl.BlockSpec((c, hid), lambda i: (0, 0)),
            pl.BlockSpec((1, hid), lambda i: (0, 0)),
            pl.BlockSpec((hid, out_dim), lambda i: (0, 0)),
            pl.BlockSpec((bm1, n), lambda i: (i, 0)),
        ],
        out_specs=(
            pl.BlockSpec((bm1, out_dim), lambda i: (i, 0)),
            pl.BlockSpec((bm1, n), lambda i: (i, 0)),
        ),
        out_shape=(
            jax.ShapeDtypeStruct((n, out_dim), _F8),
            jax.ShapeDtypeStruct((n, n), _F8),
        ),
        scratch_shapes=[pltpu.VMEM((n, hid), jnp.bfloat16)],
    )(x, W1, b1.reshape(1, -1), W2, adj)

    out = pl.pallas_call(
        _layer2_kernel,
        grid=(n // bm2,),
        in_specs=[
            pl.BlockSpec((n, out_dim), lambda i: (0, 0)),
            pl.BlockSpec((1, out_dim), lambda i: (0, 0)),
            pl.BlockSpec((bm2, n), lambda i: (i, 0)),
        ],
        out_specs=pl.BlockSpec((bm2, out_dim), lambda i: (i, 0)),
        out_shape=jax.ShapeDtypeStruct((n, out_dim), jnp.float32),
        compiler_params=pltpu.CompilerParams(vmem_limit_bytes=64 * 1024 * 1024),
    )(hw2s, b2.reshape(1, -1), qadj)
    return out


# speedup vs baseline: 1.0248x; 1.0248x over previous
import jax
import jax.numpy as jnp
from jax.experimental import pallas as pl
from jax.experimental.pallas import tpu as pltpu

_F8 = jnp.float8_e4m3fn


def _layer1_kernel(x_ref, w1_ref, b1_ref, w2_ref, adj_ref,
                   hw2_ref, qadj_ref, xw1_scr):
    @pl.when(pl.program_id(0) == 0)
    def _():
        xw1 = jnp.dot(
            x_ref[...].astype(jnp.bfloat16),
            w1_ref[...].astype(jnp.bfloat16),
            preferred_element_type=jnp.float32,
        )
        xw1_scr[...] = xw1.astype(jnp.bfloat16)

    a = adj_ref[...]
    qadj_ref[...] = a.astype(_F8)

    h = (
        jnp.dot(
            a.astype(jnp.bfloat16),
            xw1_scr[...],
            preferred_element_type=jnp.float32,
        )
        + b1_ref[...]
    )
    h = jnp.maximum(h, 0.0)
    hw2s = jnp.dot(
        h.astype(jnp.bfloat16),
        w2_ref[...].astype(jnp.bfloat16),
        preferred_element_type=jnp.float32,
    )
    hw2_ref[...] = hw2s.astype(_F8)


def _layer2_kernel(hw2_ref, b2_ref, qadj_ref, out_ref):
    out_ref[...] = (
        jnp.dot(
            qadj_ref[...],
            hw2_ref[...],
            preferred_element_type=jnp.float32,
        )
        + b2_ref[...]
    )


def kernel(x, adj, W1, b1, W2, b2):
    n, c = x.shape
    hid = W1.shape[1]
    out_dim = W2.shape[1]
    bm1 = 400
    bm2 = 1000

    hw2s, qadj = pl.pallas_call(
        _layer1_kernel,
        grid=(n // bm1,),
        in_specs=[
            pl.BlockSpec((n, c), lambda i: (0, 0)),
            pl.BlockSpec((c, hid), lambda i: (0, 0)),
            pl.BlockSpec((1, hid), lambda i: (0, 0)),
            pl.BlockSpec((hid, out_dim), lambda i: (0, 0)),
            pl.BlockSpec((bm1, n), lambda i: (i, 0)),
        ],
        out_specs=(
            pl.BlockSpec((bm1, out_dim), lambda i: (i, 0)),
            pl.BlockSpec((bm1, n), lambda i: (i, 0)),
        ),
        out_shape=(
            jax.ShapeDtypeStruct((n, out_dim), _F8),
            jax.ShapeDtypeStruct((n, n), _F8),
        ),
        scratch_shapes=[pltpu.VMEM((n, hid), jnp.bfloat16)],
    )(x, W1, b1.reshape(1, -1), W2, adj)

    out = pl.pallas_call(
        _layer2_kernel,
        grid=(n // bm2,),
        in_specs=[
            pl.BlockSpec((n, out_dim), lambda i: (0, 0)),
            pl.BlockSpec((1, out_dim), lambda i: (0, 0)),
            pl.BlockSpec((bm2, n), lambda i: (i, 0)),
        ],
        out_specs=pl.BlockSpec((bm2, out_dim), lambda i: (i, 0)),
        out_shape=jax.ShapeDtypeStruct((n, out_dim), jnp.float32),
    )(hw2s, b2.reshape(1, -1), qadj)
    return out
